# Initial kernel scaffold; baseline (speedup 1.0000x reference)
import jax
import jax.numpy as jnp
from jax.experimental import pallas as pl


def kernel(pos, batch, cA, bA, cB, bB, c1, b1, c2, b2):
    raise NotImplementedError("write your pallas kernel here")


# speedup vs baseline: 2.6585x; 2.6585x over previous
import functools

import numpy as np
import jax
import jax.numpy as jnp
from jax import lax
from jax.experimental import pallas as pl
from jax.experimental.pallas import tpu as pltpu
from jax.experimental.pallas import tpu_sc as plsc

_N = 8192
_NG = 8
_K = 20
_G = 8
_EMB = 1024
_OUT = 40
_F32 = jnp.float32

_TOPK_ROWS = 128
_EDGE_T = 256
_NW = 32
_BPW = (_N * _K) // _NW


def _wf_np():
    n = np.arange(4, dtype=np.float64)
    w = 0.5 * (1.0 - np.cos(2.0 * np.pi * n / 3.0))
    wf = np.zeros(8, dtype=np.float64)
    for j in range(3):
        wf[2 * j:2 * j + 4] += w
    return wf.astype(np.float32)


def _fold(c, wf):
    cw = c * jnp.asarray(wf)
    out = c.shape[1]
    wc = jnp.transpose(cw[0], (2, 1, 0)).reshape(-1, out)
    ws = jnp.transpose(cw[1], (2, 1, 0)).reshape(-1, out)
    return jnp.concatenate([wc, ws], axis=0)


def _feat(x):
    e = jnp.concatenate([x * float(g + 1) for g in range(_G)], axis=1)
    return jnp.concatenate([jnp.cos(e), jnp.sin(e)], axis=1)


def _topk_kernel(posr_ref, posT_ref, bcol_ref, brow_ref, idx_ref, d_ref):
    acc = None
    for c in range(3):
        t = posr_ref[:, c:c + 1] - posT_ref[c:c + 1, :]
        sq = t * t
        acc = sq if acc is None else acc + sq
    mask = bcol_ref[:, :] != brow_ref[:, :]
    d_ref[:, :] = jnp.where(mask, jnp.float32(3.0e38), acc)
    d = d_ref[:, :]
    ii = lax.broadcasted_iota(jnp.int32, (_TOPK_ROWS, _N), 1)
    prev_d = jnp.full((_TOPK_ROWS, 1), -1.0, _F32)
    prev_i = jnp.full((_TOPK_ROWS, 1), -1, jnp.int32)
    for k in range(_K):
        valid = (d > prev_d) | ((d == prev_d) & (ii > prev_i))
        dk = jnp.min(jnp.where(valid, d, jnp.inf), axis=1, keepdims=True)
        sel = valid & (d == dk)
        ik = jnp.min(jnp.where(sel, ii, _N), axis=1, keepdims=True)
        idx_ref[:, k:k + 1] = ik
        prev_d, prev_i = dk, ik


def _knn_topk(pos_pad8, posT8, batch_col, batch_row):
    grid = (_N // _TOPK_ROWS,)
    return pl.pallas_call(
        _topk_kernel,
        grid=grid,
        in_specs=[
            pl.BlockSpec((_TOPK_ROWS, 8), lambda t: (t, 0)),
            pl.BlockSpec((8, _N), lambda t: (0, 0)),
            pl.BlockSpec((_TOPK_ROWS, 1), lambda t: (t, 0)),
            pl.BlockSpec((1, _N), lambda t: (0, 0)),
        ],
        out_specs=pl.BlockSpec((_TOPK_ROWS, _K), lambda t: (t, 0)),
        out_shape=jax.ShapeDtypeStruct((_N, _K), jnp.int32),
        scratch_shapes=[pltpu.VMEM((_TOPK_ROWS, _N), _F32)],
    )(pos_pad8, posT8, batch_col, batch_row)


def _sc_gather(table16, idx_flat):
    mesh = plsc.VectorSubcoreMesh(core_axis_name="c", subcore_axis_name="s")

    @functools.partial(
        pl.kernel,
        mesh=mesh,
        compiler_params=pltpu.CompilerParams(use_tc_tiling_on_sc=False),
        out_type=jax.ShapeDtypeStruct((_K * _N, 16), _F32),
        scratch_types=[
            pltpu.VMEM((_BPW,), jnp.int32),
            pltpu.VMEM((_BPW, 16), _F32),
            pltpu.SemaphoreType.DMA,
        ],
    )
    def gk(table_hbm, idx_hbm, out_hbm, idx_v, rows_v, sem):
        wid = lax.axis_index("s") * 2 + lax.axis_index("c")
        base = wid * _BPW
        pltpu.sync_copy(idx_hbm.at[pl.ds(base, _BPW)], idx_v)
        pltpu.async_copy(table_hbm.at[idx_v], rows_v, sem).wait()
        pltpu.sync_copy(rows_v, out_hbm.at[pl.ds(base, _BPW)])

    return gk(table16, idx_flat)


def _edge_kernel(posj_ref, pos_ref, bcol_ref, wa_ref, ba_ref, wb_ref, bb_ref,
                 w1_ref, b1_ref, xmax_ref, xsum_ref, cnt_ref):
    t = pl.program_id(0)
    pi3 = pos_ref[:, 0:3]
    wa = wa_ref[:, :]
    ba = ba_ref[:, :]
    wb = wb_ref[:, :]
    bb = bb_ref[:, :]

    def body(k, acc):
        pj = jnp.reshape(posj_ref[pl.ds(k, 1), :, :], (_EDGE_T, 16))
        e = jnp.concatenate([pi3, pj[:, 0:3] - pi3], axis=1)
        ha = jnp.dot(_feat(e), wa, preferred_element_type=_F32) + ba
        hb = jnp.dot(_feat(ha), wb, preferred_element_type=_F32) + bb
        return jnp.maximum(acc, hb)

    h = lax.fori_loop(0, _K, body, jnp.full((_EDGE_T, 128), -jnp.inf, _F32))
    x = jnp.dot(_feat(h), w1_ref[:, :], preferred_element_type=_F32) + b1_ref[:, :]

    @pl.when(t == 0)
    def _init():
        xmax_ref[:, :] = jnp.full_like(xmax_ref, -jnp.inf)
        xsum_ref[:, :] = jnp.zeros_like(xsum_ref)
        cnt_ref[:, :] = jnp.zeros_like(cnt_ref)

    bvals = bcol_ref[:, :]
    for g in range(_NG):
        m = bvals == g
        xg = jnp.max(jnp.where(m, x, -jnp.inf), axis=0, keepdims=True)
        sg = jnp.sum(jnp.where(m, x, 0.0), axis=0, keepdims=True)
        cg = jnp.sum(m.astype(_F32))
        xmax_ref[g:g + 1, :] = jnp.maximum(xmax_ref[g:g + 1, :], xg)
        xsum_ref[g:g + 1, :] = xsum_ref[g:g + 1, :] + sg
        cnt_ref[g:g + 1, :] = cnt_ref[g:g + 1, :] + cg


def _edge_call(posj3, pos_pad16, batch_col, WA, bA2, WB, bB2, W1, b12):
    grid = (_N // _EDGE_T,)
    return pl.pallas_call(
        _edge_kernel,
        grid=grid,
        in_specs=[
            pl.BlockSpec((_K, _EDGE_T, 16), lambda t: (0, t, 0)),
            pl.BlockSpec((_EDGE_T, 16), lambda t: (t, 0)),
            pl.BlockSpec((_EDGE_T, 1), lambda t: (t, 0)),
            pl.BlockSpec((96, 64), lambda t: (0, 0)),
            pl.BlockSpec((1, 64), lambda t: (0, 0)),
            pl.BlockSpec((1024, 128), lambda t: (0, 0)),
            pl.BlockSpec((1, 128), lambda t: (0, 0)),
            pl.BlockSpec((2048, _EMB), lambda t: (0, 0)),
            pl.BlockSpec((1, _EMB), lambda t: (0, 0)),
        ],
        out_specs=[
            pl.BlockSpec((_NG, _EMB), lambda t: (0, 0)),
            pl.BlockSpec((_NG, _EMB), lambda t: (0, 0)),
            pl.BlockSpec((_NG, 128), lambda t: (0, 0)),
        ],
        out_shape=[
            jax.ShapeDtypeStruct((_NG, _EMB), _F32),
            jax.ShapeDtypeStruct((_NG, _EMB), _F32),
            jax.ShapeDtypeStruct((_NG, 128), _F32),
        ],
    )(posj3, pos_pad16, batch_col, WA, bA2, WB, bB2, W1, b12)


def _head_kernel(xmax_ref, xsum_ref, cnt_ref, w2_ref, b2_ref, out_ref):
    cntv = cnt_ref[:, 0:1]
    xmean = xsum_ref[:, :] / jnp.maximum(cntv, 1.0)
    xcat = jnp.concatenate([xmax_ref[:, :], xmean], axis=1)
    out_ref[:, :] = (jnp.dot(_feat(xcat), w2_ref[:, :],
                             preferred_element_type=_F32) + b2_ref[:, :])


def _head_call(xmax, xsum, cnt, W2, b22):
    return pl.pallas_call(
        _head_kernel,
        out_shape=jax.ShapeDtypeStruct((_NG, _OUT), _F32),
    )(xmax, xsum, cnt, W2, b22)


def kernel(pos, batch, cA, bA, cB, bB, c1, b1, c2, b2):
    wf = _wf_np()
    WA = _fold(cA, wf)
    WB = _fold(cB, wf)
    W1 = _fold(c1, wf)
    W2 = _fold(c2, wf)

    pos_pad8 = jnp.pad(pos, ((0, 0), (0, 5)))
    pos_pad16 = jnp.pad(pos, ((0, 0), (0, 13)))
    posT8 = pos_pad8.T
    batch_col = batch.reshape(_N, 1)
    batch_row = batch.reshape(1, _N)

    idx = _knn_topk(pos_pad8, posT8, batch_col, batch_row)
    idx_flat = idx.T.reshape(-1)
    posj = _sc_gather(pos_pad16, idx_flat)
    posj3 = posj.reshape(_K, _N, 16)

    xmax, xsum, cnt = _edge_call(
        posj3, pos_pad16, batch_col, WA, bA.reshape(1, -1), WB,
        bB.reshape(1, -1), W1, b1.reshape(1, -1))
    return _head_call(xmax, xsum, cnt, W2, b2.reshape(1, -1))


# speedup vs baseline: 4.9734x; 1.8708x over previous
import functools

import numpy as np
import jax
import jax.numpy as jnp
from jax import lax
from jax.experimental import pallas as pl
from jax.experimental.pallas import tpu as pltpu
from jax.experimental.pallas import tpu_sc as plsc

_N = 8192
_NG = 8
_K = 20
_G = 8
_EMB = 1024
_OUT = 40
_F32 = jnp.float32

_TOPK_ROWS = 128
_EDGE_T = 256
_NW = 32
_BPW = (_N * _K) // _NW


def _wf_np():
    n = np.arange(4, dtype=np.float64)
    w = 0.5 * (1.0 - np.cos(2.0 * np.pi * n / 3.0))
    wf = np.zeros(8, dtype=np.float64)
    for j in range(3):
        wf[2 * j:2 * j + 4] += w
    return wf.astype(np.float32)


def _fold(c, wf):
    cw = c * jnp.asarray(wf)
    out = c.shape[1]
    wc = jnp.transpose(cw[0], (2, 1, 0)).reshape(-1, out)
    ws = jnp.transpose(cw[1], (2, 1, 0)).reshape(-1, out)
    return jnp.concatenate([wc, ws], axis=0)


def _feat(x):
    e = jnp.concatenate([x * float(g + 1) for g in range(_G)], axis=1)
    return jnp.concatenate([jnp.cos(e), jnp.sin(e)], axis=1)


def _feat_rec(x):
    c1, s1 = jnp.cos(x), jnp.sin(x)
    cs, ss = [c1], [s1]
    for _ in range(_G - 1):
        cn = cs[-1] * c1 - ss[-1] * s1
        sn = ss[-1] * c1 + cs[-1] * s1
        cs.append(cn)
        ss.append(sn)
    return jnp.concatenate(cs + ss, axis=1)


def _topk_kernel(posr_ref, posT_ref, bcol_ref, brow_ref, idx_ref, d_ref):
    acc = None
    for c in range(3):
        t = posr_ref[:, c:c + 1] - posT_ref[c:c + 1, :]
        sq = t * t
        acc = sq if acc is None else acc + sq
    mask = bcol_ref[:, :] != brow_ref[:, :]
    d_ref[:, :] = jnp.where(mask, jnp.float32(3.0e38), acc)
    ii = lax.broadcasted_iota(jnp.int32, (_TOPK_ROWS, _N), 1)
    for k in range(_K):
        d = d_ref[:, :]
        dk = jnp.min(d, axis=1, keepdims=True)
        ik = jnp.min(jnp.where(d == dk, ii, _N), axis=1, keepdims=True)
        idx_ref[:, k:k + 1] = ik
        d_ref[:, :] = jnp.where(ii == ik, jnp.inf, d)


def _knn_topk(pos_pad8, posT8, batch_col, batch_row):
    grid = (_N // _TOPK_ROWS,)
    return pl.pallas_call(
        _topk_kernel,
        grid=grid,
        in_specs=[
            pl.BlockSpec((_TOPK_ROWS, 8), lambda t: (t, 0)),
            pl.BlockSpec((8, _N), lambda t: (0, 0)),
            pl.BlockSpec((_TOPK_ROWS, 1), lambda t: (t, 0)),
            pl.BlockSpec((1, _N), lambda t: (0, 0)),
        ],
        out_specs=pl.BlockSpec((_TOPK_ROWS, _K), lambda t: (t, 0)),
        out_shape=jax.ShapeDtypeStruct((_N, _K), jnp.int32),
        scratch_shapes=[pltpu.VMEM((_TOPK_ROWS, _N), _F32)],
    )(pos_pad8, posT8, batch_col, batch_row)


def _sc_gather(table16, idx_flat):
    mesh = plsc.VectorSubcoreMesh(core_axis_name="c", subcore_axis_name="s")

    @functools.partial(
        pl.kernel,
        mesh=mesh,
        compiler_params=pltpu.CompilerParams(use_tc_tiling_on_sc=False),
        out_type=jax.ShapeDtypeStruct((_K * _N, 16), _F32),
        scratch_types=[
            pltpu.VMEM((_BPW,), jnp.int32),
            pltpu.VMEM((_BPW, 16), _F32),
            pltpu.SemaphoreType.DMA,
        ],
    )
    def gk(table_hbm, idx_hbm, out_hbm, idx_v, rows_v, sem):
        wid = lax.axis_index("s") * 2 + lax.axis_index("c")
        base = wid * _BPW
        pltpu.sync_copy(idx_hbm.at[pl.ds(base, _BPW)], idx_v)
        pltpu.async_copy(table_hbm.at[idx_v], rows_v, sem).wait()
        pltpu.sync_copy(rows_v, out_hbm.at[pl.ds(base, _BPW)])

    return gk(table16, idx_flat)


def _edge_kernel(posj_ref, pos_ref, bcol_ref, wa_ref, ba_ref, wb_ref, bb_ref,
                 w1_ref, b1_ref, xmax_ref, xsum_ref, cnt_ref):
    t = pl.program_id(0)
    pi3 = pos_ref[:, 0:3]
    wa = wa_ref[:, :]
    ba = ba_ref[:, :]
    wb = wb_ref[:, :]
    bb = bb_ref[:, :]

    epi = jnp.concatenate([pi3 * float(g + 1) for g in range(_G)], axis=1)
    cpi, spi = jnp.cos(epi), jnp.sin(epi)

    def body(k, acc):
        pj = jnp.reshape(posj_ref[pl.ds(k, 1), :, :], (_EDGE_T, 16))
        dj = pj[:, 0:3] - pi3
        edj = jnp.concatenate([dj * float(g + 1) for g in range(_G)], axis=1)
        fa = jnp.concatenate([cpi, jnp.cos(edj), spi, jnp.sin(edj)], axis=1)
        ha = jnp.dot(fa, wa, preferred_element_type=_F32) + ba
        hb = jnp.dot(_feat_rec(ha), wb, preferred_element_type=_F32) + bb
        return jnp.maximum(acc, hb)

    h = lax.fori_loop(0, _K, body, jnp.full((_EDGE_T, 128), -jnp.inf, _F32))
    x = (jnp.dot(_feat_rec(h), w1_ref[:, :], preferred_element_type=_F32)
         + b1_ref[:, :])

    @pl.when(t == 0)
    def _init():
        xmax_ref[:, :] = jnp.full_like(xmax_ref, -jnp.inf)
        xsum_ref[:, :] = jnp.zeros_like(xsum_ref)
        cnt_ref[:, :] = jnp.zeros_like(cnt_ref)

    bvals = bcol_ref[:, :]
    for g in range(_NG):
        m = bvals == g
        xg = jnp.max(jnp.where(m, x, -jnp.inf), axis=0, keepdims=True)
        sg = jnp.sum(jnp.where(m, x, 0.0), axis=0, keepdims=True)
        cg = jnp.sum(m.astype(_F32))
        xmax_ref[g:g + 1, :] = jnp.maximum(xmax_ref[g:g + 1, :], xg)
        xsum_ref[g:g + 1, :] = xsum_ref[g:g + 1, :] + sg
        cnt_ref[g:g + 1, :] = cnt_ref[g:g + 1, :] + cg


def _edge_call(posj3, pos_pad16, batch_col, WA, bA2, WB, bB2, W1, b12):
    grid = (_N // _EDGE_T,)
    return pl.pallas_call(
        _edge_kernel,
        grid=grid,
        in_specs=[
            pl.BlockSpec((_K, _EDGE_T, 16), lambda t: (0, t, 0)),
            pl.BlockSpec((_EDGE_T, 16), lambda t: (t, 0)),
            pl.BlockSpec((_EDGE_T, 1), lambda t: (t, 0)),
            pl.BlockSpec((96, 64), lambda t: (0, 0)),
            pl.BlockSpec((1, 64), lambda t: (0, 0)),
            pl.BlockSpec((1024, 128), lambda t: (0, 0)),
            pl.BlockSpec((1, 128), lambda t: (0, 0)),
            pl.BlockSpec((2048, _EMB), lambda t: (0, 0)),
            pl.BlockSpec((1, _EMB), lambda t: (0, 0)),
        ],
        out_specs=[
            pl.BlockSpec((_NG, _EMB), lambda t: (0, 0)),
            pl.BlockSpec((_NG, _EMB), lambda t: (0, 0)),
            pl.BlockSpec((_NG, 128), lambda t: (0, 0)),
        ],
        out_shape=[
            jax.ShapeDtypeStruct((_NG, _EMB), _F32),
            jax.ShapeDtypeStruct((_NG, _EMB), _F32),
            jax.ShapeDtypeStruct((_NG, 128), _F32),
        ],
    )(posj3, pos_pad16, batch_col, WA, bA2, WB, bB2, W1, b12)


def _head_kernel(xmax_ref, xsum_ref, cnt_ref, w2_ref, b2_ref, out_ref):
    cntv = cnt_ref[:, 0:1]
    xmean = xsum_ref[:, :] / jnp.maximum(cntv, 1.0)
    xcat = jnp.concatenate([xmax_ref[:, :], xmean], axis=1)
    out_ref[:, :] = (jnp.dot(_feat(xcat), w2_ref[:, :],
                             preferred_element_type=_F32) + b2_ref[:, :])


def _head_call(xmax, xsum, cnt, W2, b22):
    return pl.pallas_call(
        _head_kernel,
        out_shape=jax.ShapeDtypeStruct((_NG, _OUT), _F32),
    )(xmax, xsum, cnt, W2, b22)


def kernel(pos, batch, cA, bA, cB, bB, c1, b1, c2, b2):
    wf = _wf_np()
    WA = _fold(cA, wf)
    permA = np.concatenate([
        np.array([t * 48 + g * 6 + i for g in range(_G) for i in rng])
        for t, rng in ((0, range(3)), (0, range(3, 6)),
                       (1, range(3)), (1, range(3, 6)))])
    WA = WA[permA]
    WB = _fold(cB, wf)
    W1 = _fold(c1, wf)
    W2 = _fold(c2, wf)

    pos_pad8 = jnp.pad(pos, ((0, 0), (0, 5)))
    pos_pad16 = jnp.pad(pos, ((0, 0), (0, 13)))
    posT8 = pos_pad8.T
    batch_col = batch.reshape(_N, 1)
    batch_row = batch.reshape(1, _N)

    idx = _knn_topk(pos_pad8, posT8, batch_col, batch_row)
    idx_flat = idx.T.reshape(-1)
    posj = _sc_gather(pos_pad16, idx_flat)
    posj3 = posj.reshape(_K, _N, 16)

    xmax, xsum, cnt = _edge_call(
        posj3, pos_pad16, batch_col, WA, bA.reshape(1, -1), WB,
        bB.reshape(1, -1), W1, b1.reshape(1, -1))
    return _head_call(xmax, xsum, cnt, W2, b2.reshape(1, -1))


# speedup vs baseline: 5.4455x; 1.0949x over previous
import functools

import numpy as np
import jax
import jax.numpy as jnp
from jax import lax
from jax.experimental import pallas as pl
from jax.experimental.pallas import tpu as pltpu
from jax.experimental.pallas import tpu_sc as plsc

_N = 8192
_NG = 8
_K = 20
_G = 8
_EMB = 1024
_OUT = 40
_F32 = jnp.float32

_TOPK_ROWS = 128
_EDGE_T = 256
_NW = 32
_BPW = (_N * _K) // _NW


def _wf_np():
    n = np.arange(4, dtype=np.float64)
    w = 0.5 * (1.0 - np.cos(2.0 * np.pi * n / 3.0))
    wf = np.zeros(8, dtype=np.float64)
    for j in range(3):
        wf[2 * j:2 * j + 4] += w
    return wf.astype(np.float32)


def _fold(c, wf):
    cw = c * jnp.asarray(wf)
    out = c.shape[1]
    wc = jnp.transpose(cw[0], (2, 1, 0)).reshape(-1, out)
    ws = jnp.transpose(cw[1], (2, 1, 0)).reshape(-1, out)
    return jnp.concatenate([wc, ws], axis=0)


def _feat(x):
    e = jnp.concatenate([x * float(g + 1) for g in range(_G)], axis=1)
    return jnp.concatenate([jnp.cos(e), jnp.sin(e)], axis=1)


def _feat_rec(x):
    d = x.shape[1]
    c1, s1 = jnp.cos(x), jnp.sin(x)
    c1p = jnp.concatenate([c1, c1], axis=1)
    s1pm = jnp.concatenate([-s1, s1], axis=1)
    z = jnp.concatenate([c1, s1], axis=1)
    zs = [z]
    for _ in range(_G - 1):
        zswap = jnp.concatenate([z[:, d:], z[:, :d]], axis=1)
        z = z * c1p + zswap * s1pm
        zs.append(z)
    return jnp.concatenate(zs, axis=1)


def _perm_gcs(indim):
    rows = []
    for g in range(_G):
        rows.extend(range(g * indim, (g + 1) * indim))
        rows.extend(range(_G * indim + g * indim, _G * indim + (g + 1) * indim))
    return np.asarray(rows)


def _topk_kernel(posr_ref, posT_ref, bcol_ref, brow_ref, idx_ref, d_ref):
    acc = None
    for c in range(3):
        t = posr_ref[:, c:c + 1] - posT_ref[c:c + 1, :]
        sq = t * t
        acc = sq if acc is None else acc + sq
    mask = bcol_ref[:, :] != brow_ref[:, :]
    d_ref[:, :] = jnp.where(mask, jnp.float32(3.0e38), acc)
    iif = lax.broadcasted_iota(jnp.int32, (_TOPK_ROWS, _N), 1).astype(_F32)
    for k in range(_K):
        d = d_ref[:, :]
        dk = jnp.min(d, axis=1, keepdims=True)
        ikf = jnp.min(jnp.where(d == dk, iif, jnp.float32(3.4e38)),
                      axis=1, keepdims=True)
        idx_ref[:, k:k + 1] = ikf.astype(jnp.int32)
        d_ref[:, :] = jnp.where(iif == ikf, jnp.inf, d)


def _knn_topk(pos_pad8, posT8, batch_col, batch_row):
    grid = (_N // _TOPK_ROWS,)
    return pl.pallas_call(
        _topk_kernel,
        grid=grid,
        in_specs=[
            pl.BlockSpec((_TOPK_ROWS, 8), lambda t: (t, 0)),
            pl.BlockSpec((8, _N), lambda t: (0, 0)),
            pl.BlockSpec((_TOPK_ROWS, 1), lambda t: (t, 0)),
            pl.BlockSpec((1, _N), lambda t: (0, 0)),
        ],
        out_specs=pl.BlockSpec((_TOPK_ROWS, _K), lambda t: (t, 0)),
        out_shape=jax.ShapeDtypeStruct((_N, _K), jnp.int32),
        scratch_shapes=[pltpu.VMEM((_TOPK_ROWS, _N), _F32)],
    )(pos_pad8, posT8, batch_col, batch_row)


def _sc_gather(table16, idx_flat):
    mesh = plsc.VectorSubcoreMesh(core_axis_name="c", subcore_axis_name="s")

    @functools.partial(
        pl.kernel,
        mesh=mesh,
        compiler_params=pltpu.CompilerParams(use_tc_tiling_on_sc=False),
        out_type=jax.ShapeDtypeStruct((_K * _N, 16), _F32),
        scratch_types=[
            pltpu.VMEM((_BPW,), jnp.int32),
            pltpu.VMEM((_BPW, 16), _F32),
            pltpu.SemaphoreType.DMA,
        ],
    )
    def gk(table_hbm, idx_hbm, out_hbm, idx_v, rows_v, sem):
        wid = lax.axis_index("s") * 2 + lax.axis_index("c")
        base = wid * _BPW
        pltpu.sync_copy(idx_hbm.at[pl.ds(base, _BPW)], idx_v)
        pltpu.async_copy(table_hbm.at[idx_v], rows_v, sem).wait()
        pltpu.sync_copy(rows_v, out_hbm.at[pl.ds(base, _BPW)])

    return gk(table16, idx_flat)


def _edge_kernel(posj_ref, pos_ref, bcol_ref, wa_ref, ba_ref, wb_ref, bb_ref,
                 w1_ref, b1_ref, xmax_ref, xsum_ref, cnt_ref):
    t = pl.program_id(0)
    pi3 = pos_ref[:, 0:3]
    wa = wa_ref[:, :]
    ba = ba_ref[:, :]
    wb = wb_ref[:, :]
    bb = bb_ref[:, :]

    epi = jnp.concatenate([pi3 * float(g + 1) for g in range(_G)], axis=1)
    cpi, spi = jnp.cos(epi), jnp.sin(epi)

    def one_k(k):
        pj = jnp.reshape(posj_ref[pl.ds(k, 1), :, :], (_EDGE_T, 16))
        dj = pj[:, 0:3] - pi3
        edj = jnp.concatenate([dj * float(g + 1) for g in range(_G)], axis=1)
        fa = jnp.concatenate([cpi, jnp.cos(edj), spi, jnp.sin(edj)], axis=1)
        ha = jnp.dot(fa, wa, preferred_element_type=_F32) + ba
        return jnp.dot(_feat_rec(ha), wb, preferred_element_type=_F32) + bb

    def body(k2, acc):
        hb0 = one_k(2 * k2)
        hb1 = one_k(2 * k2 + 1)
        return jnp.maximum(acc, jnp.maximum(hb0, hb1))

    h = lax.fori_loop(0, _K // 2, body,
                      jnp.full((_EDGE_T, 128), -jnp.inf, _F32))
    x = (jnp.dot(_feat_rec(h), w1_ref[:, :], preferred_element_type=_F32)
         + b1_ref[:, :])

    @pl.when(t == 0)
    def _init():
        xmax_ref[:, :] = jnp.full_like(xmax_ref, -jnp.inf)
        xsum_ref[:, :] = jnp.zeros_like(xsum_ref)
        cnt_ref[:, :] = jnp.zeros_like(cnt_ref)

    bvals = bcol_ref[:, :]
    for g in range(_NG):
        m = bvals == g
        xg = jnp.max(jnp.where(m, x, -jnp.inf), axis=0, keepdims=True)
        sg = jnp.sum(jnp.where(m, x, 0.0), axis=0, keepdims=True)
        cg = jnp.sum(m.astype(_F32))
        xmax_ref[g:g + 1, :] = jnp.maximum(xmax_ref[g:g + 1, :], xg)
        xsum_ref[g:g + 1, :] = xsum_ref[g:g + 1, :] + sg
        cnt_ref[g:g + 1, :] = cnt_ref[g:g + 1, :] + cg


def _edge_call(posj3, pos_pad16, batch_col, WA, bA2, WB, bB2, W1, b12):
    grid = (_N // _EDGE_T,)
    return pl.pallas_call(
        _edge_kernel,
        grid=grid,
        in_specs=[
            pl.BlockSpec((_K, _EDGE_T, 16), lambda t: (0, t, 0)),
            pl.BlockSpec((_EDGE_T, 16), lambda t: (t, 0)),
            pl.BlockSpec((_EDGE_T, 1), lambda t: (t, 0)),
            pl.BlockSpec((96, 64), lambda t: (0, 0)),
            pl.BlockSpec((1, 64), lambda t: (0, 0)),
            pl.BlockSpec((1024, 128), lambda t: (0, 0)),
            pl.BlockSpec((1, 128), lambda t: (0, 0)),
            pl.BlockSpec((2048, _EMB), lambda t: (0, 0)),
            pl.BlockSpec((1, _EMB), lambda t: (0, 0)),
        ],
        out_specs=[
            pl.BlockSpec((_NG, _EMB), lambda t: (0, 0)),
            pl.BlockSpec((_NG, _EMB), lambda t: (0, 0)),
            pl.BlockSpec((_NG, 128), lambda t: (0, 0)),
        ],
        out_shape=[
            jax.ShapeDtypeStruct((_NG, _EMB), _F32),
            jax.ShapeDtypeStruct((_NG, _EMB), _F32),
            jax.ShapeDtypeStruct((_NG, 128), _F32),
        ],
    )(posj3, pos_pad16, batch_col, WA, bA2, WB, bB2, W1, b12)


def _head_kernel(xmax_ref, xsum_ref, cnt_ref, w2_ref, b2_ref, out_ref):
    cntv = cnt_ref[:, 0:1]
    xmean = xsum_ref[:, :] / jnp.maximum(cntv, 1.0)
    xcat = jnp.concatenate([xmax_ref[:, :], xmean], axis=1)
    out_ref[:, :] = (jnp.dot(_feat(xcat), w2_ref[:, :],
                             preferred_element_type=_F32) + b2_ref[:, :])


def _head_call(xmax, xsum, cnt, W2, b22):
    return pl.pallas_call(
        _head_kernel,
        out_shape=jax.ShapeDtypeStruct((_NG, _OUT), _F32),
    )(xmax, xsum, cnt, W2, b22)


def kernel(pos, batch, cA, bA, cB, bB, c1, b1, c2, b2):
    wf = _wf_np()
    WA = _fold(cA, wf)
    permA = np.concatenate([
        np.array([t * 48 + g * 6 + i for g in range(_G) for i in rng])
        for t, rng in ((0, range(3)), (0, range(3, 6)),
                       (1, range(3)), (1, range(3, 6)))])
    WA = WA[permA]
    WB = _fold(cB, wf)[_perm_gcs(64)]
    W1 = _fold(c1, wf)[_perm_gcs(128)]
    W2 = _fold(c2, wf)

    pos_pad8 = jnp.pad(pos, ((0, 0), (0, 5)))
    pos_pad16 = jnp.pad(pos, ((0, 0), (0, 13)))
    posT8 = pos_pad8.T
    batch_col = batch.reshape(_N, 1)
    batch_row = batch.reshape(1, _N)

    idx = _knn_topk(pos_pad8, posT8, batch_col, batch_row)
    idx_flat = idx.T.reshape(-1)
    posj = _sc_gather(pos_pad16, idx_flat)
    posj3 = posj.reshape(_K, _N, 16)

    xmax, xsum, cnt = _edge_call(
        posj3, pos_pad16, batch_col, WA, bA.reshape(1, -1), WB,
        bB.reshape(1, -1), W1, b1.reshape(1, -1))
    return _head_call(xmax, xsum, cnt, W2, b2.reshape(1, -1))


# speedup vs baseline: 6.0423x; 1.1096x over previous
import functools

import numpy as np
import jax
import jax.numpy as jnp
from jax import lax
from jax.experimental import pallas as pl
from jax.experimental.pallas import tpu as pltpu
from jax.experimental.pallas import tpu_sc as plsc

_N = 8192
_NG = 8
_K = 20
_G = 8
_EMB = 1024
_OUT = 40
_F32 = jnp.float32

_TOPK_ROWS = 128
_EDGE_T = 256
_NW = 32
_BPW = (_N * _K) // _NW


def _wf_np():
    n = np.arange(4, dtype=np.float64)
    w = 0.5 * (1.0 - np.cos(2.0 * np.pi * n / 3.0))
    wf = np.zeros(8, dtype=np.float64)
    for j in range(3):
        wf[2 * j:2 * j + 4] += w
    return wf.astype(np.float32)


def _fold(c, wf):
    cw = c * jnp.asarray(wf)
    out = c.shape[1]
    wc = jnp.transpose(cw[0], (2, 1, 0)).reshape(-1, out)
    ws = jnp.transpose(cw[1], (2, 1, 0)).reshape(-1, out)
    return jnp.concatenate([wc, ws], axis=0)


def _feat(x):
    e = jnp.concatenate([x * float(g + 1) for g in range(_G)], axis=1)
    return jnp.concatenate([jnp.cos(e), jnp.sin(e)], axis=1)


def _rec_from(c1, s1):
    d = c1.shape[1]
    c1p = jnp.concatenate([c1, c1], axis=1)
    s1pm = jnp.concatenate([-s1, s1], axis=1)
    z = jnp.concatenate([c1, s1], axis=1)
    zs = [z]
    for _ in range(_G - 1):
        zswap = jnp.concatenate([z[:, d:], z[:, :d]], axis=1)
        z = z * c1p + zswap * s1pm
        zs.append(z)
    return jnp.concatenate(zs, axis=1)


def _feat_rec(x):
    return _rec_from(jnp.cos(x), jnp.sin(x))


def _perm_gcs(indim):
    rows = []
    for g in range(_G):
        rows.extend(range(g * indim, (g + 1) * indim))
        rows.extend(range(_G * indim + g * indim, _G * indim + (g + 1) * indim))
    return np.asarray(rows)


def _topk_kernel(posr_ref, posT_ref, bcol_ref, brow_ref, idx_ref, d_ref):
    acc = None
    for c in range(3):
        t = posr_ref[:, c:c + 1] - posT_ref[c:c + 1, :]
        sq = t * t
        acc = sq if acc is None else acc + sq
    mask = bcol_ref[:, :] != brow_ref[:, :]
    d_ref[:, :] = jnp.where(mask, jnp.float32(3.0e38), acc)
    iif = lax.broadcasted_iota(jnp.int32, (_TOPK_ROWS, _N), 1).astype(_F32)
    for k in range(_K):
        d = d_ref[:, :]
        dk = jnp.min(d, axis=1, keepdims=True)
        ikf = jnp.min(jnp.where(d == dk, iif, jnp.float32(3.4e38)),
                      axis=1, keepdims=True)
        idx_ref[:, k:k + 1] = ikf.astype(jnp.int32)
        d_ref[:, :] = jnp.where(iif == ikf, jnp.inf, d)


def _knn_topk(pos_pad8, posT8, batch_col, batch_row):
    grid = (_N // _TOPK_ROWS,)
    return pl.pallas_call(
        _topk_kernel,
        grid=grid,
        in_specs=[
            pl.BlockSpec((_TOPK_ROWS, 8), lambda t: (t, 0)),
            pl.BlockSpec((8, _N), lambda t: (0, 0)),
            pl.BlockSpec((_TOPK_ROWS, 1), lambda t: (t, 0)),
            pl.BlockSpec((1, _N), lambda t: (0, 0)),
        ],
        out_specs=pl.BlockSpec((_TOPK_ROWS, _K), lambda t: (t, 0)),
        out_shape=jax.ShapeDtypeStruct((_N, _K), jnp.int32),
        scratch_shapes=[pltpu.VMEM((_TOPK_ROWS, _N), _F32)],
    )(pos_pad8, posT8, batch_col, batch_row)


def _sc_gather(table16, idx_flat):
    mesh = plsc.VectorSubcoreMesh(core_axis_name="c", subcore_axis_name="s")

    @functools.partial(
        pl.kernel,
        mesh=mesh,
        compiler_params=pltpu.CompilerParams(use_tc_tiling_on_sc=False),
        out_type=jax.ShapeDtypeStruct((_K * _N, 16), _F32),
        scratch_types=[
            pltpu.VMEM((_BPW,), jnp.int32),
            pltpu.VMEM((_BPW, 16), _F32),
            pltpu.SemaphoreType.DMA,
        ],
    )
    def gk(table_hbm, idx_hbm, out_hbm, idx_v, rows_v, sem):
        wid = lax.axis_index("s") * 2 + lax.axis_index("c")
        base = wid * _BPW
        pltpu.sync_copy(idx_hbm.at[pl.ds(base, _BPW)], idx_v)
        pltpu.async_copy(table_hbm.at[idx_v], rows_v, sem).wait()
        pltpu.sync_copy(rows_v, out_hbm.at[pl.ds(base, _BPW)])

    return gk(table16, idx_flat)


def _edge_kernel(posj_ref, pos_ref, bcol_ref, wa_ref, ba_ref, wb_ref, bb_ref,
                 w1_ref, b1_ref, xmax_ref, xsum_ref, cnt_ref):
    t = pl.program_id(0)
    pi3 = pos_ref[:, 0:3]
    wa = wa_ref[:, :]
    ba = ba_ref[:, :]
    wb = wb_ref[:, :]
    bb = bb_ref[:, :]

    epi = jnp.concatenate([pi3 * float(g + 1) for g in range(_G)], axis=1)
    cpi, spi = jnp.cos(epi), jnp.sin(epi)

    def body(k2, acc):
        pj0 = jnp.reshape(posj_ref[pl.ds(2 * k2, 1), :, :], (_EDGE_T, 16))
        pj1 = jnp.reshape(posj_ref[pl.ds(2 * k2 + 1, 1), :, :], (_EDGE_T, 16))
        dj0 = pj0[:, 0:3] - pi3
        dj1 = pj1[:, 0:3] - pi3
        edj = jnp.concatenate(
            [dj0 * float(g + 1) for g in range(_G)]
            + [dj1 * float(g + 1) for g in range(_G)], axis=1)
        ce, se = jnp.cos(edj), jnp.sin(edj)
        fa = jnp.concatenate(
            [jnp.concatenate([cpi, ce[:, 24 * j:24 * (j + 1)], spi,
                              se[:, 24 * j:24 * (j + 1)]], axis=1)
             for j in range(2)], axis=0)
        ha = jnp.dot(fa, wa, preferred_element_type=_F32) + ba
        hal = jnp.concatenate([ha[:_EDGE_T], ha[_EDGE_T:]], axis=1)
        cl, sl = jnp.cos(hal), jnp.sin(hal)
        fb = jnp.concatenate(
            [_rec_from(cl[:, 64 * j:64 * (j + 1)],
                       sl[:, 64 * j:64 * (j + 1)]) for j in range(2)], axis=0)
        hb = jnp.dot(fb, wb, preferred_element_type=_F32) + bb
        return jnp.maximum(acc, jnp.maximum(hb[:_EDGE_T], hb[_EDGE_T:]))

    h = lax.fori_loop(0, _K // 2, body,
                      jnp.full((_EDGE_T, 128), -jnp.inf, _F32))
    x = (jnp.dot(_feat_rec(h), w1_ref[:, :], preferred_element_type=_F32)
         + b1_ref[:, :])

    @pl.when(t == 0)
    def _init():
        xmax_ref[:, :] = jnp.full_like(xmax_ref, -jnp.inf)
        xsum_ref[:, :] = jnp.zeros_like(xsum_ref)
        cnt_ref[:, :] = jnp.zeros_like(cnt_ref)

    bvals = bcol_ref[:, :]
    for g in range(_NG):
        m = bvals == g
        xg = jnp.max(jnp.where(m, x, -jnp.inf), axis=0, keepdims=True)
        sg = jnp.sum(jnp.where(m, x, 0.0), axis=0, keepdims=True)
        cg = jnp.sum(m.astype(_F32))
        xmax_ref[g:g + 1, :] = jnp.maximum(xmax_ref[g:g + 1, :], xg)
        xsum_ref[g:g + 1, :] = xsum_ref[g:g + 1, :] + sg
        cnt_ref[g:g + 1, :] = cnt_ref[g:g + 1, :] + cg


def _edge_call(posj3, pos_pad16, batch_col, WA, bA2, WB, bB2, W1, b12):
    grid = (_N // _EDGE_T,)
    return pl.pallas_call(
        _edge_kernel,
        grid=grid,
        in_specs=[
            pl.BlockSpec((_K, _EDGE_T, 16), lambda t: (0, t, 0)),
            pl.BlockSpec((_EDGE_T, 16), lambda t: (t, 0)),
            pl.BlockSpec((_EDGE_T, 1), lambda t: (t, 0)),
            pl.BlockSpec((96, 64), lambda t: (0, 0)),
            pl.BlockSpec((1, 64), lambda t: (0, 0)),
            pl.BlockSpec((1024, 128), lambda t: (0, 0)),
            pl.BlockSpec((1, 128), lambda t: (0, 0)),
            pl.BlockSpec((2048, _EMB), lambda t: (0, 0)),
            pl.BlockSpec((1, _EMB), lambda t: (0, 0)),
        ],
        out_specs=[
            pl.BlockSpec((_NG, _EMB), lambda t: (0, 0)),
            pl.BlockSpec((_NG, _EMB), lambda t: (0, 0)),
            pl.BlockSpec((_NG, 128), lambda t: (0, 0)),
        ],
        out_shape=[
            jax.ShapeDtypeStruct((_NG, _EMB), _F32),
            jax.ShapeDtypeStruct((_NG, _EMB), _F32),
            jax.ShapeDtypeStruct((_NG, 128), _F32),
        ],
    )(posj3, pos_pad16, batch_col, WA, bA2, WB, bB2, W1, b12)


def _head_kernel(xmax_ref, xsum_ref, cnt_ref, w2_ref, b2_ref, out_ref):
    cntv = cnt_ref[:, 0:1]
    xmean = xsum_ref[:, :] / jnp.maximum(cntv, 1.0)
    xcat = jnp.concatenate([xmax_ref[:, :], xmean], axis=1)
    out_ref[:, :] = (jnp.dot(_feat(xcat), w2_ref[:, :],
                             preferred_element_type=_F32) + b2_ref[:, :])


def _head_call(xmax, xsum, cnt, W2, b22):
    return pl.pallas_call(
        _head_kernel,
        out_shape=jax.ShapeDtypeStruct((_NG, _OUT), _F32),
    )(xmax, xsum, cnt, W2, b22)


def kernel(pos, batch, cA, bA, cB, bB, c1, b1, c2, b2):
    wf = _wf_np()
    WA = _fold(cA, wf)
    permA = np.concatenate([
        np.array([t * 48 + g * 6 + i for g in range(_G) for i in rng])
        for t, rng in ((0, range(3)), (0, range(3, 6)),
                       (1, range(3)), (1, range(3, 6)))])
    WA = WA[permA]
    WB = _fold(cB, wf)[_perm_gcs(64)]
    W1 = _fold(c1, wf)[_perm_gcs(128)]
    W2 = _fold(c2, wf)

    pos_pad8 = jnp.pad(pos, ((0, 0), (0, 5)))
    pos_pad16 = jnp.pad(pos, ((0, 0), (0, 13)))
    posT8 = pos_pad8.T
    batch_col = batch.reshape(_N, 1)
    batch_row = batch.reshape(1, _N)

    idx = _knn_topk(pos_pad8, posT8, batch_col, batch_row)
    idx_flat = idx.T.reshape(-1)
    posj = _sc_gather(pos_pad16, idx_flat)
    posj3 = posj.reshape(_K, _N, 16)

    xmax, xsum, cnt = _edge_call(
        posj3, pos_pad16, batch_col, WA, bA.reshape(1, -1), WB,
        bB.reshape(1, -1), W1, b1.reshape(1, -1))
    return _head_call(xmax, xsum, cnt, W2, b2.reshape(1, -1))
